# baseline (speedup 1.0000x reference)
import functools

import jax
import jax.numpy as jnp
from jax import lax
from jax.experimental import pallas as pl
from jax.experimental.pallas import tpu as pltpu
from jax.experimental.pallas import tpu_sc as plsc

MASK = -1000000000.0
TAU = 0.5
N = 10000
NE = 320000
D = 128
NPAD = 10240
NT = 32
EPT = NE // NT
K = 125
NCHUNK = EPT // K
RB = 1280
GB = NPAD // RB
SLAB = NPAD // 16

CH = D // 2
KK = 128
NBUF = 4
ECHUNKS = 160
NEP = 16 * ECHUNKS * KK
TR = 128
ACCR = NPAD + TR
ZSLAB = ACCR // 16
OSLAB = NPAD // 16

_mesh = plsc.VectorSubcoreMesh(core_axis_name="c", subcore_axis_name="s")
_sc_params = pltpu.CompilerParams(use_tc_tiling_on_sc=False,
                                  needs_layout_passes=False)


@functools.partial(
    pl.kernel,
    out_type=jax.ShapeDtypeStruct((2, NPAD), jnp.float32),
    mesh=_mesh,
    scratch_types=[
        pltpu.VMEM((NCHUNK, K), jnp.int32),
        pltpu.VMEM((SLAB,), jnp.float32),
        pltpu.VMEM((128,), jnp.float32),
        pltpu.VMEM_SHARED((NPAD,), jnp.float32),
    ],
    compiler_params=_sc_params,
)
def _sc_degree(dst_hbm, out_hbm, dst_v, stage, ones_v, acc):
    cid = lax.axis_index("c")
    sid = lax.axis_index("s")
    tid = sid * 2 + cid
    z16 = jnp.zeros((16,), jnp.float32)
    o16 = jnp.full((16,), 1.0, jnp.float32)
    for i in range(SLAB // 16):
        stage[pl.ds(i * 16, 16)] = z16
    for i in range(8):
        ones_v[pl.ds(i * 16, 16)] = o16
    pltpu.sync_copy(stage, acc.at[pl.ds(sid * SLAB, SLAB)])
    pltpu.sync_copy(dst_hbm.at[tid], dst_v)
    plsc.subcore_barrier()

    def body(j, c):
        pltpu.sync_copy(ones_v.at[pl.ds(0, K)], acc.at[dst_v.at[j]], add=True)
        return c

    lax.fori_loop(0, NCHUNK, body, 0)
    plsc.subcore_barrier()
    pltpu.sync_copy(acc.at[pl.ds(sid * SLAB, SLAB)], stage)
    pltpu.sync_copy(stage, out_hbm.at[cid].at[pl.ds(sid * SLAB, SLAB)])


@functools.partial(
    pl.kernel,
    out_type=jax.ShapeDtypeStruct((2 * NPAD, CH), jnp.float32),
    mesh=_mesh,
    scratch_types=[
        pltpu.VMEM((ECHUNKS, KK), jnp.int32),
        pltpu.VMEM((ECHUNKS, KK), jnp.int32),
        pltpu.VMEM((OSLAB // 128, 128), jnp.int32),
        pltpu.VMEM((NBUF, KK, CH), jnp.float32),
        pltpu.VMEM((128, CH), jnp.float32),
        pltpu.VMEM_SHARED((ACCR, CH), jnp.float32),
        [pltpu.SemaphoreType.DMA] * NBUF,
        [pltpu.SemaphoreType.DMA] * NBUF,
    ],
    compiler_params=_sc_params,
)
def _sc_edge_acc(g2_hbm, src_hbm, dst_hbm, oidx_hbm, out_hbm,
                 src_v, dst_v, oidx_v, rows, zb, acc, gsem, ssem):
    cid = lax.axis_index("c")
    sid = lax.axis_index("s")
    z16 = jnp.zeros((16,), jnp.float32)

    def zbody(r, c):
        for jj in range(CH // 16):
            zb[r, pl.ds(jj * 16, 16)] = z16
        return c

    lax.fori_loop(0, 128, zbody, 0)
    for kk in range(ZSLAB // 128):
        pltpu.sync_copy(zb, acc.at[pl.ds(sid * ZSLAB + kk * 128, 128)])
    pltpu.sync_copy(zb.at[pl.ds(0, ZSLAB % 128)],
                    acc.at[pl.ds(sid * ZSLAB + (ZSLAB // 128) * 128,
                                 ZSLAB % 128)])
    pltpu.sync_copy(src_hbm.at[cid].at[sid], src_v)
    pltpu.sync_copy(dst_hbm.at[sid], dst_v)
    pltpu.sync_copy(oidx_hbm.at[cid].at[sid], oidx_v)
    plsc.subcore_barrier()

    gh = g2_hbm
    pltpu.async_copy(gh.at[src_v.at[0]], rows.at[0], gsem[0])
    pltpu.async_copy(gh.at[src_v.at[1]], rows.at[1], gsem[1])

    def body(r, c):
        for k in range(NBUF):
            j = r * NBUF + k
            b = k
            bn = (k + 2) % NBUF
            pltpu.make_async_copy(gh.at[src_v.at[j]], rows.at[b],
                                  gsem[b]).wait()
            pltpu.async_copy(rows.at[b], acc.at[dst_v.at[j]], ssem[b],
                             add=True)

            @pl.when(j >= 2)
            def _():
                pltpu.make_async_copy(rows.at[bn], acc.at[dst_v.at[j - 2]],
                                      ssem[bn]).wait()

            @pl.when(j + 2 < ECHUNKS)
            def _():
                pltpu.async_copy(gh.at[src_v.at[j + 2]], rows.at[bn],
                                 gsem[bn])
        return c

    lax.fori_loop(0, ECHUNKS // NBUF, body, 0)
    pltpu.make_async_copy(rows.at[2], acc.at[dst_v.at[ECHUNKS - 2]],
                          ssem[2]).wait()
    pltpu.make_async_copy(rows.at[3], acc.at[dst_v.at[ECHUNKS - 1]],
                          ssem[3]).wait()
    plsc.subcore_barrier()
    for kk in range(OSLAB // 128):
        pltpu.sync_copy(acc.at[pl.ds(sid * OSLAB + kk * 128, 128)], zb)
        pltpu.sync_copy(zb, out_hbm.at[oidx_v.at[kk]])


@functools.partial(
    pl.kernel,
    out_type=jax.ShapeDtypeStruct((NT, N), jnp.float32),
    mesh=_mesh,
    scratch_types=[
        pltpu.VMEM((EPT,), jnp.int32),
        pltpu.VMEM((EPT,), jnp.int32),
        pltpu.VMEM((16,), jnp.int32),
        pltpu.VMEM((NPAD,), jnp.float32),
    ],
    compiler_params=_sc_params,
)
def _sc_conn(srcf_hbm, dstf_hbm, pi_hbm, out_hbm, src_v, dst_v, pi_v, hist):
    cid = lax.axis_index("c")
    sid = lax.axis_index("s")
    tid = sid * 2 + cid
    z16 = jnp.zeros((16,), jnp.float32)
    o16 = jnp.full((16,), 1.0, jnp.float32)

    def zb(i, c):
        hist[pl.ds(i * 16, 16)] = z16
        return c

    lax.fori_loop(0, NPAD // 16, zb, 0)
    pltpu.sync_copy(srcf_hbm.at[tid], src_v)
    pltpu.sync_copy(dstf_hbm.at[tid], dst_v)
    pltpu.sync_copy(pi_hbm, pi_v)
    pi16 = pi_v[...]

    def body(i, c):
        s16 = src_v[pl.ds(i * 16, 16)]
        d16 = dst_v[pl.ds(i * 16, 16)]
        m = (s16 == pi16) | (d16 == pi16)
        plsc.store_scatter(hist, [s16], o16, mask=m)
        plsc.store_scatter(hist, [d16], o16, mask=m)
        return c

    lax.fori_loop(0, EPT // 16, body, 0)
    pltpu.sync_copy(hist.at[pl.ds(0, N)], out_hbm.at[tid])


def _tca_body(x_ref, w_ref, degb_ref, u_ref, dis_ref):
    dis = lax.rsqrt(degb_ref[...])
    dis_ref[...] = dis
    u_ref[...] = dis * jnp.dot(x_ref[...], w_ref[...],
                               preferred_element_type=jnp.float32)


def _comb_body(p_ref, u_ref, dis_ref, b_ref, w_ref, uo_ref):
    t = jnp.maximum(
        dis_ref[...] * (p_ref[...] + u_ref[...]) + b_ref[...], 0.0)
    uo_ref[...] = dis_ref[...] * jnp.dot(t, w_ref[...],
                                         preferred_element_type=jnp.float32)


def _d1_body(p_ref, u_ref, dis_ref, b_ref, gcn_ref, pooled_ref):
    i = pl.program_id(0)
    g = jnp.maximum(
        dis_ref[...] * (p_ref[...] + u_ref[...]) + b_ref[...], 0.0)
    rid = lax.broadcasted_iota(jnp.int32, (RB, D), 0) + i * RB
    valid = rid < N
    g = jnp.where(valid, g, 0.0)
    gcn_ref[...] = g
    s = jnp.sum(g.reshape(RB // 8, 8, D), axis=0)
    mx = jnp.max(jnp.where(valid, g, -3.4e38).reshape(RB // 8, 8, D), axis=0)

    @pl.when(i == 0)
    def _():
        pooled_ref[0] = s
        pooled_ref[1] = mx

    @pl.when(i > 0)
    def _():
        pooled_ref[0] = pooled_ref[0] + s
        pooled_ref[1] = jnp.maximum(pooled_ref[1], mx)


def _d2_body(pooled_ref, wn1_ref, wn2_ref, n1_ref, n2_ref):
    mean = jnp.sum(pooled_ref[0], axis=0, keepdims=True) * (1.0 / N)
    mx = jnp.max(pooled_ref[1], axis=0, keepdims=True)
    gp = jnp.concatenate([mean, mx], axis=1)
    n1_ref[...] = jnp.dot(gp, wn1_ref[...],
                          preferred_element_type=jnp.float32)
    n2_ref[...] = jnp.dot(gp, wn2_ref[...],
                          preferred_element_type=jnp.float32)


def _d3_body(n1_ref, bn1_ref, g1_ref, s1_ref, pi_ref):
    col = lax.broadcasted_iota(jnp.int32, (1, N), 1)
    z = (n1_ref[...] + bn1_ref[...] + g1_ref[...]) * (1.0 / TAU)
    m = jnp.max(z, axis=1, keepdims=True)
    e = jnp.exp(z - m)
    s1_ref[...] = e / jnp.sum(e, axis=1, keepdims=True)
    idx = jnp.min(jnp.where(z >= m, col, jnp.int32(2 ** 30)))
    pi_ref[...] = jnp.zeros((8, 128), jnp.int32) + idx


def _e_body(connp_ref, n2_ref, bn2_ref, g2_ref, pi_ref, gcn_ref, wep_ref,
            bep_ref, g3_ref, s2_ref, et_ref, stop_ref):
    pi = pi_ref[0, 0]
    col = lax.broadcasted_iota(jnp.int32, (1, N), 1)
    csum = jnp.sum(connp_ref[...], axis=0, keepdims=True)
    conn = (csum > 0.0) | (col == pi)
    logit = jnp.where(conn, MASK, n2_ref[...] + bn2_ref[...])
    z = (logit + g2_ref[...]) * (1.0 / TAU)
    m = jnp.max(z, axis=1, keepdims=True)
    e = jnp.exp(z - m)
    s2_ref[...] = e / jnp.sum(e, axis=1, keepdims=True)
    i2 = jnp.min(jnp.where(z >= m, col, jnp.int32(2 ** 30)))
    r1 = gcn_ref[pl.ds(pi, 1), :]
    r2 = gcn_ref[pl.ds(i2, 1), :]
    emb = jnp.concatenate([r1, r2], axis=1)
    etl = jnp.dot(emb, wep_ref[...],
                  preferred_element_type=jnp.float32) + bep_ref[...] + g3_ref[...]
    c3 = lax.broadcasted_iota(jnp.int32, (1, 128), 1)
    z3 = jnp.where(c3 < 3, etl * (1.0 / TAU), -3.4e38)
    m3 = jnp.max(z3, axis=1, keepdims=True)
    e3 = jnp.exp(z3 - m3)
    et_ref[...] = e3 / jnp.sum(e3, axis=1, keepdims=True)
    stop_ref[...] = jnp.zeros((8, 128), jnp.float32) + 1.0


def _rowspec(i_map=lambda i: (i, 0)):
    return pl.BlockSpec((RB, D), i_map)


def kernel(x, edge_index, W1, b1, W2, b2, W3, b3,
           Wn1, bn1, Wn2, bn2, We, be, Ws, bs):
    del Ws, bs
    f32 = jnp.float32
    src = edge_index[0]
    dst = edge_index[1]
    dst3 = dst.reshape(NT, NCHUNK, K)
    srcf = src.reshape(NT, EPT)
    dstf = dst.reshape(NT, EPT)
    pe = NEP - NE
    spread = jnp.arange(pe, dtype=jnp.int32)
    s2 = 2 * jnp.concatenate([src, spread]).reshape(16, ECHUNKS, KK)
    srcp2 = jnp.stack([s2, s2 + 1])
    trash = NPAD + (jnp.arange(pe, dtype=jnp.int32) & (TR - 1))
    dstp = jnp.concatenate([dst, trash]).reshape(16, ECHUNKS, KK)
    rb = (jnp.arange(16, dtype=jnp.int32)[:, None, None] * OSLAB
          + jnp.arange(OSLAB // 128, dtype=jnp.int32)[None, :, None] * 128
          + jnp.arange(128, dtype=jnp.int32)[None, None, :])
    oidx = jnp.stack([2 * rb, 2 * rb + 1])
    xp = jnp.pad(x, ((0, NPAD - N), (0, 0)))

    def edge_acc(u):
        out = _sc_edge_acc(u.reshape(2 * NPAD, CH), srcp2, dstp, oidx)
        return out.reshape(NPAD, D)

    k1, k2, k3, _ = jax.random.split(jax.random.key(42), 4)

    def gum(k, shape):
        u = jax.random.uniform(k, shape, minval=1e-10, maxval=1.0)
        return -jnp.log(-jnp.log(u))

    g1p = gum(k1, (1, N))
    g2p = gum(k2, (1, N))
    g3p = jnp.pad(gum(k3, (1, 3)), ((0, 0), (0, 125)))

    b1r = b1.reshape(1, D)
    b2r = b2.reshape(1, D)
    b3r = b3.reshape(1, D)
    bn1p = bn1.reshape(1, N)
    bn2p = bn2.reshape(1, N)
    wep = jnp.pad(We, ((0, 0), (0, 125)))
    ber = jnp.pad(be, (0, 125)).reshape(1, 128)

    degp = _sc_degree(dst3)
    degb = jnp.broadcast_to((degp[0] + degp[1] + 1.0)[:, None], (NPAD, D))

    u1, disf = pl.pallas_call(
        _tca_body,
        grid=(GB,),
        in_specs=[_rowspec(), pl.BlockSpec((D, D), lambda i: (0, 0)),
                  _rowspec()],
        out_specs=[_rowspec(), _rowspec()],
        out_shape=[jax.ShapeDtypeStruct((NPAD, D), f32)] * 2,
    )(xp, W1, degb)

    def comb(p, u, bias, w):
        return pl.pallas_call(
            _comb_body,
            grid=(GB,),
            in_specs=[_rowspec(), _rowspec(), _rowspec(),
                      pl.BlockSpec((1, D), lambda i: (0, 0)),
                      pl.BlockSpec((D, D), lambda i: (0, 0))],
            out_specs=_rowspec(),
            out_shape=jax.ShapeDtypeStruct((NPAD, D), f32),
        )(p, u, disf, bias, w)

    p1 = edge_acc(u1)
    u2 = comb(p1, u1, b1r, W2)
    p2 = edge_acc(u2)
    u3 = comb(p2, u2, b2r, W3)
    p3 = edge_acc(u3)

    gcn, pooled = pl.pallas_call(
        _d1_body,
        grid=(GB,),
        in_specs=[_rowspec(), _rowspec(), _rowspec(),
                  pl.BlockSpec((1, D), lambda i: (0, 0))],
        out_specs=[_rowspec(), pl.BlockSpec((2, 8, D), lambda i: (0, 0, 0))],
        out_shape=[jax.ShapeDtypeStruct((NPAD, D), f32),
                   jax.ShapeDtypeStruct((2, 8, D), f32)],
    )(p3, u3, disf, b3r)

    n1, n2 = pl.pallas_call(
        _d2_body,
        out_shape=[jax.ShapeDtypeStruct((1, N), f32)] * 2,
    )(pooled, Wn1, Wn2)

    s1, piarr = pl.pallas_call(
        _d3_body,
        out_shape=[jax.ShapeDtypeStruct((1, N), f32),
                   jax.ShapeDtypeStruct((8, 128), jnp.int32)],
    )(n1, bn1p, g1p)

    pi16 = piarr[0, :16]
    connp = _sc_conn(srcf, dstf, pi16)

    s2, et, stop = pl.pallas_call(
        _e_body,
        out_shape=[jax.ShapeDtypeStruct((1, N), f32),
                   jax.ShapeDtypeStruct((1, 128), f32),
                   jax.ShapeDtypeStruct((8, 128), f32)],
    )(connp, n2, bn2p, g2p, piarr, gcn, wep, ber, g3p)

    return (s1, s2, et[:, :3], stop[:1, :1])


# speedup vs baseline: 1.0078x; 1.0078x over previous
import functools

import jax
import jax.numpy as jnp
from jax import lax
from jax.experimental import pallas as pl
from jax.experimental.pallas import tpu as pltpu
from jax.experimental.pallas import tpu_sc as plsc

MASK = -1000000000.0
TAU = 0.5
N = 10000
NE = 320000
D = 128
NPAD = 10240
NT = 32
EPT = NE // NT
K = 125
NCHUNK = EPT // K
RB = 1280
GB = NPAD // RB
SLAB = NPAD // 16

CH = D // 2
KK = 128
NBUF = 4
ECHUNKS = 160
NEP = 16 * ECHUNKS * KK
TR = 128
ACCR = NPAD + TR
ZSLAB = ACCR // 16
OSLAB = NPAD // 16

_mesh = plsc.VectorSubcoreMesh(core_axis_name="c", subcore_axis_name="s")
_sc_params = pltpu.CompilerParams(use_tc_tiling_on_sc=False,
                                  needs_layout_passes=False)


@functools.partial(
    pl.kernel,
    out_type=jax.ShapeDtypeStruct((2, NPAD), jnp.float32),
    mesh=_mesh,
    scratch_types=[
        pltpu.VMEM((NCHUNK, K), jnp.int32),
        pltpu.VMEM((SLAB,), jnp.float32),
        pltpu.VMEM((128,), jnp.float32),
        pltpu.VMEM_SHARED((NPAD,), jnp.float32),
    ],
    compiler_params=_sc_params,
)
def _sc_degree(dst_hbm, out_hbm, dst_v, stage, ones_v, acc):
    cid = lax.axis_index("c")
    sid = lax.axis_index("s")
    tid = sid * 2 + cid
    z16 = jnp.zeros((16,), jnp.float32)
    o16 = jnp.full((16,), 1.0, jnp.float32)
    for i in range(SLAB // 16):
        stage[pl.ds(i * 16, 16)] = z16
    for i in range(8):
        ones_v[pl.ds(i * 16, 16)] = o16
    pltpu.sync_copy(stage, acc.at[pl.ds(sid * SLAB, SLAB)])
    pltpu.sync_copy(dst_hbm.at[tid], dst_v)
    plsc.subcore_barrier()

    def body(j, c):
        pltpu.sync_copy(ones_v.at[pl.ds(0, K)], acc.at[dst_v.at[j]], add=True)
        return c

    lax.fori_loop(0, NCHUNK, body, 0)
    plsc.subcore_barrier()
    pltpu.sync_copy(acc.at[pl.ds(sid * SLAB, SLAB)], stage)
    pltpu.sync_copy(stage, out_hbm.at[cid].at[pl.ds(sid * SLAB, SLAB)])


@functools.partial(
    pl.kernel,
    out_type=jax.ShapeDtypeStruct((2 * NPAD, CH), jnp.float32),
    mesh=_mesh,
    scratch_types=[
        pltpu.VMEM((ECHUNKS, KK), jnp.int32),
        pltpu.VMEM((ECHUNKS, KK), jnp.int32),
        pltpu.VMEM((OSLAB // 128, 128), jnp.int32),
        pltpu.VMEM((NBUF, KK, CH), jnp.float32),
        pltpu.VMEM((128, CH), jnp.float32),
        pltpu.VMEM_SHARED((ACCR, CH), jnp.float32),
        [pltpu.SemaphoreType.DMA] * NBUF,
        [pltpu.SemaphoreType.DMA] * NBUF,
    ],
    compiler_params=_sc_params,
)
def _sc_edge_acc(g2_hbm, src_hbm, dst_hbm, oidx_hbm, out_hbm,
                 src_v, dst_v, oidx_v, rows, zb, acc, gsem, ssem):
    cid = lax.axis_index("c")
    sid = lax.axis_index("s")
    z16 = jnp.zeros((16,), jnp.float32)

    pltpu.sync_copy(src_hbm.at[cid].at[sid], src_v)
    pltpu.sync_copy(dst_hbm.at[sid], dst_v)
    pltpu.sync_copy(oidx_hbm.at[cid].at[sid], oidx_v)
    gh = g2_hbm
    pltpu.async_copy(gh.at[src_v.at[0]], rows.at[0], gsem[0])
    pltpu.async_copy(gh.at[src_v.at[1]], rows.at[1], gsem[1])

    def zbody(r, c):
        for jj in range(CH // 16):
            zb[r, pl.ds(jj * 16, 16)] = z16
        return c

    lax.fori_loop(0, 128, zbody, 0)
    for kk in range(ZSLAB // 128):
        pltpu.sync_copy(zb, acc.at[pl.ds(sid * ZSLAB + kk * 128, 128)])
    pltpu.sync_copy(zb.at[pl.ds(0, ZSLAB % 128)],
                    acc.at[pl.ds(sid * ZSLAB + (ZSLAB // 128) * 128,
                                 ZSLAB % 128)])
    plsc.subcore_barrier()


    def body(r, c):
        for k in range(NBUF):
            j = r * NBUF + k
            b = k
            bn = (k + 2) % NBUF
            pltpu.make_async_copy(gh.at[src_v.at[j]], rows.at[b],
                                  gsem[b]).wait()
            pltpu.async_copy(rows.at[b], acc.at[dst_v.at[j]], ssem[b],
                             add=True)

            @pl.when(j >= 2)
            def _():
                pltpu.make_async_copy(rows.at[bn], acc.at[dst_v.at[j - 2]],
                                      ssem[bn]).wait()

            @pl.when(j + 2 < ECHUNKS)
            def _():
                pltpu.async_copy(gh.at[src_v.at[j + 2]], rows.at[bn],
                                 gsem[bn])
        return c

    lax.fori_loop(0, ECHUNKS // NBUF, body, 0)
    pltpu.make_async_copy(rows.at[2], acc.at[dst_v.at[ECHUNKS - 2]],
                          ssem[2]).wait()
    pltpu.make_async_copy(rows.at[3], acc.at[dst_v.at[ECHUNKS - 1]],
                          ssem[3]).wait()
    plsc.subcore_barrier()
    for kk in range(OSLAB // 128):
        pltpu.sync_copy(acc.at[pl.ds(sid * OSLAB + kk * 128, 128)], zb)
        pltpu.sync_copy(zb, out_hbm.at[oidx_v.at[kk]])


@functools.partial(
    pl.kernel,
    out_type=jax.ShapeDtypeStruct((NT, N), jnp.float32),
    mesh=_mesh,
    scratch_types=[
        pltpu.VMEM((EPT,), jnp.int32),
        pltpu.VMEM((EPT,), jnp.int32),
        pltpu.VMEM((16,), jnp.int32),
        pltpu.VMEM((NPAD,), jnp.float32),
    ],
    compiler_params=_sc_params,
)
def _sc_conn(srcf_hbm, dstf_hbm, pi_hbm, out_hbm, src_v, dst_v, pi_v, hist):
    cid = lax.axis_index("c")
    sid = lax.axis_index("s")
    tid = sid * 2 + cid
    z16 = jnp.zeros((16,), jnp.float32)
    o16 = jnp.full((16,), 1.0, jnp.float32)

    def zb(i, c):
        hist[pl.ds(i * 16, 16)] = z16
        return c

    lax.fori_loop(0, NPAD // 16, zb, 0)
    pltpu.sync_copy(srcf_hbm.at[tid], src_v)
    pltpu.sync_copy(dstf_hbm.at[tid], dst_v)
    pltpu.sync_copy(pi_hbm, pi_v)
    pi16 = pi_v[...]

    def body(i, c):
        s16 = src_v[pl.ds(i * 16, 16)]
        d16 = dst_v[pl.ds(i * 16, 16)]
        m = (s16 == pi16) | (d16 == pi16)
        plsc.store_scatter(hist, [s16], o16, mask=m)
        plsc.store_scatter(hist, [d16], o16, mask=m)
        return c

    lax.fori_loop(0, EPT // 16, body, 0)
    pltpu.sync_copy(hist.at[pl.ds(0, N)], out_hbm.at[tid])


def _tca_body(x_ref, w_ref, degb_ref, u_ref, dis_ref):
    dis = lax.rsqrt(degb_ref[...])
    dis_ref[...] = dis
    u_ref[...] = dis * jnp.dot(x_ref[...], w_ref[...],
                               preferred_element_type=jnp.float32)


def _comb_body(p_ref, u_ref, dis_ref, b_ref, w_ref, uo_ref):
    t = jnp.maximum(
        dis_ref[...] * (p_ref[...] + u_ref[...]) + b_ref[...], 0.0)
    uo_ref[...] = dis_ref[...] * jnp.dot(t, w_ref[...],
                                         preferred_element_type=jnp.float32)


def _d1_body(p_ref, u_ref, dis_ref, b_ref, gcn_ref, pooled_ref):
    i = pl.program_id(0)
    g = jnp.maximum(
        dis_ref[...] * (p_ref[...] + u_ref[...]) + b_ref[...], 0.0)
    rid = lax.broadcasted_iota(jnp.int32, (RB, D), 0) + i * RB
    valid = rid < N
    g = jnp.where(valid, g, 0.0)
    gcn_ref[...] = g
    s = jnp.sum(g.reshape(RB // 8, 8, D), axis=0)
    mx = jnp.max(jnp.where(valid, g, -3.4e38).reshape(RB // 8, 8, D), axis=0)

    @pl.when(i == 0)
    def _():
        pooled_ref[0] = s
        pooled_ref[1] = mx

    @pl.when(i > 0)
    def _():
        pooled_ref[0] = pooled_ref[0] + s
        pooled_ref[1] = jnp.maximum(pooled_ref[1], mx)


def _d2_body(pooled_ref, wn1_ref, wn2_ref, n1_ref, n2_ref):
    mean = jnp.sum(pooled_ref[0], axis=0, keepdims=True) * (1.0 / N)
    mx = jnp.max(pooled_ref[1], axis=0, keepdims=True)
    gp = jnp.concatenate([mean, mx], axis=1)
    n1_ref[...] = jnp.dot(gp, wn1_ref[...],
                          preferred_element_type=jnp.float32)
    n2_ref[...] = jnp.dot(gp, wn2_ref[...],
                          preferred_element_type=jnp.float32)


def _d3_body(n1_ref, bn1_ref, g1_ref, s1_ref, pi_ref):
    col = lax.broadcasted_iota(jnp.int32, (1, N), 1)
    z = (n1_ref[...] + bn1_ref[...] + g1_ref[...]) * (1.0 / TAU)
    m = jnp.max(z, axis=1, keepdims=True)
    e = jnp.exp(z - m)
    s1_ref[...] = e / jnp.sum(e, axis=1, keepdims=True)
    idx = jnp.min(jnp.where(z >= m, col, jnp.int32(2 ** 30)))
    pi_ref[...] = jnp.zeros((8, 128), jnp.int32) + idx


def _e_body(connp_ref, n2_ref, bn2_ref, g2_ref, pi_ref, gcn_ref, wep_ref,
            bep_ref, g3_ref, s2_ref, et_ref, stop_ref):
    pi = pi_ref[0, 0]
    col = lax.broadcasted_iota(jnp.int32, (1, N), 1)
    csum = jnp.sum(connp_ref[...], axis=0, keepdims=True)
    conn = (csum > 0.0) | (col == pi)
    logit = jnp.where(conn, MASK, n2_ref[...] + bn2_ref[...])
    z = (logit + g2_ref[...]) * (1.0 / TAU)
    m = jnp.max(z, axis=1, keepdims=True)
    e = jnp.exp(z - m)
    s2_ref[...] = e / jnp.sum(e, axis=1, keepdims=True)
    i2 = jnp.min(jnp.where(z >= m, col, jnp.int32(2 ** 30)))
    r1 = gcn_ref[pl.ds(pi, 1), :]
    r2 = gcn_ref[pl.ds(i2, 1), :]
    emb = jnp.concatenate([r1, r2], axis=1)
    etl = jnp.dot(emb, wep_ref[...],
                  preferred_element_type=jnp.float32) + bep_ref[...] + g3_ref[...]
    c3 = lax.broadcasted_iota(jnp.int32, (1, 128), 1)
    z3 = jnp.where(c3 < 3, etl * (1.0 / TAU), -3.4e38)
    m3 = jnp.max(z3, axis=1, keepdims=True)
    e3 = jnp.exp(z3 - m3)
    et_ref[...] = e3 / jnp.sum(e3, axis=1, keepdims=True)
    stop_ref[...] = jnp.zeros((8, 128), jnp.float32) + 1.0


def _rowspec(i_map=lambda i: (i, 0)):
    return pl.BlockSpec((RB, D), i_map)


def kernel(x, edge_index, W1, b1, W2, b2, W3, b3,
           Wn1, bn1, Wn2, bn2, We, be, Ws, bs):
    del Ws, bs
    f32 = jnp.float32
    src = edge_index[0]
    dst = edge_index[1]
    dst3 = dst.reshape(NT, NCHUNK, K)
    srcf = src.reshape(NT, EPT)
    dstf = dst.reshape(NT, EPT)
    pe = NEP - NE
    spread = jnp.arange(pe, dtype=jnp.int32)
    s2 = 2 * jnp.concatenate([src, spread]).reshape(16, ECHUNKS, KK)
    srcp2 = jnp.stack([s2, s2 + 1])
    trash = NPAD + (jnp.arange(pe, dtype=jnp.int32) & (TR - 1))
    dstp = jnp.concatenate([dst, trash]).reshape(16, ECHUNKS, KK)
    rb = (jnp.arange(16, dtype=jnp.int32)[:, None, None] * OSLAB
          + jnp.arange(OSLAB // 128, dtype=jnp.int32)[None, :, None] * 128
          + jnp.arange(128, dtype=jnp.int32)[None, None, :])
    oidx = jnp.stack([2 * rb, 2 * rb + 1])
    xp = jnp.pad(x, ((0, NPAD - N), (0, 0)))

    def edge_acc(u):
        out = _sc_edge_acc(u.reshape(2 * NPAD, CH), srcp2, dstp, oidx)
        return out.reshape(NPAD, D)

    k1, k2, k3, _ = jax.random.split(jax.random.key(42), 4)

    def gum(k, shape):
        u = jax.random.uniform(k, shape, minval=1e-10, maxval=1.0)
        return -jnp.log(-jnp.log(u))

    g1p = gum(k1, (1, N))
    g2p = gum(k2, (1, N))
    g3p = jnp.pad(gum(k3, (1, 3)), ((0, 0), (0, 125)))

    b1r = b1.reshape(1, D)
    b2r = b2.reshape(1, D)
    b3r = b3.reshape(1, D)
    bn1p = bn1.reshape(1, N)
    bn2p = bn2.reshape(1, N)
    wep = jnp.pad(We, ((0, 0), (0, 125)))
    ber = jnp.pad(be, (0, 125)).reshape(1, 128)

    degp = _sc_degree(dst3)
    degb = jnp.broadcast_to((degp[0] + degp[1] + 1.0)[:, None], (NPAD, D))

    u1, disf = pl.pallas_call(
        _tca_body,
        grid=(GB,),
        in_specs=[_rowspec(), pl.BlockSpec((D, D), lambda i: (0, 0)),
                  _rowspec()],
        out_specs=[_rowspec(), _rowspec()],
        out_shape=[jax.ShapeDtypeStruct((NPAD, D), f32)] * 2,
    )(xp, W1, degb)

    def comb(p, u, bias, w):
        return pl.pallas_call(
            _comb_body,
            grid=(GB,),
            in_specs=[_rowspec(), _rowspec(), _rowspec(),
                      pl.BlockSpec((1, D), lambda i: (0, 0)),
                      pl.BlockSpec((D, D), lambda i: (0, 0))],
            out_specs=_rowspec(),
            out_shape=jax.ShapeDtypeStruct((NPAD, D), f32),
        )(p, u, disf, bias, w)

    p1 = edge_acc(u1)
    u2 = comb(p1, u1, b1r, W2)
    p2 = edge_acc(u2)
    u3 = comb(p2, u2, b2r, W3)
    p3 = edge_acc(u3)

    gcn, pooled = pl.pallas_call(
        _d1_body,
        grid=(GB,),
        in_specs=[_rowspec(), _rowspec(), _rowspec(),
                  pl.BlockSpec((1, D), lambda i: (0, 0))],
        out_specs=[_rowspec(), pl.BlockSpec((2, 8, D), lambda i: (0, 0, 0))],
        out_shape=[jax.ShapeDtypeStruct((NPAD, D), f32),
                   jax.ShapeDtypeStruct((2, 8, D), f32)],
    )(p3, u3, disf, b3r)

    n1, n2 = pl.pallas_call(
        _d2_body,
        out_shape=[jax.ShapeDtypeStruct((1, N), f32)] * 2,
    )(pooled, Wn1, Wn2)

    s1, piarr = pl.pallas_call(
        _d3_body,
        out_shape=[jax.ShapeDtypeStruct((1, N), f32),
                   jax.ShapeDtypeStruct((8, 128), jnp.int32)],
    )(n1, bn1p, g1p)

    pi16 = piarr[0, :16]
    connp = _sc_conn(srcf, dstf, pi16)

    s2, et, stop = pl.pallas_call(
        _e_body,
        out_shape=[jax.ShapeDtypeStruct((1, N), f32),
                   jax.ShapeDtypeStruct((1, 128), f32),
                   jax.ShapeDtypeStruct((8, 128), f32)],
    )(connp, n2, bn2p, g2p, piarr, gcn, wep, ber, g3p)

    return (s1, s2, et[:, :3], stop[:1, :1])
